# baseline (speedup 1.0000x reference)
import functools

import jax
import jax.numpy as jnp
from jax import lax
from jax.experimental import pallas as pl
from jax.experimental.pallas import tpu as pltpu
from jax.experimental.pallas import tpu_sc as plsc

_LANES = 16
_CHUNK = 8
_RING = 8
_DIST = 3


def _embed_kernel(n_batch, seq_len, d_model, n_workers, n_cores):
    pos_per_w = seq_len // n_workers
    n_chunks = pos_per_w // _CHUNK
    vregs_per_row = d_model // _LANES
    assert n_chunks % 2 == 0 and n_batch == 4

    mesh = plsc.VectorSubcoreMesh(core_axis_name="c", subcore_axis_name="s")

    @functools.partial(
        pl.kernel,
        mesh=mesh,
        out_type=jax.ShapeDtypeStruct((n_batch * seq_len, d_model), jnp.float32),
        scratch_types=[
            pltpu.VMEM((n_batch, pos_per_w), jnp.int32),
            pltpu.VMEM((_RING, _CHUNK, d_model), jnp.float32),
            pltpu.VMEM((2, _CHUNK, d_model), jnp.float32),
        ]
        + [pltpu.SemaphoreType.DMA] * (2 * _RING + 2),
    )
    def k(ids_hbm, tok_hbm, pos_hbm, out_hbm, idx_v, tbuf, pbuf, *sems):
        gsem = sems[:_RING]
        osem = sems[_RING:2 * _RING]
        psem = sems[2 * _RING:]
        wid = lax.axis_index("s") * n_cores + lax.axis_index("c")
        pbase = wid * pos_per_w

        pltpu.sync_copy(
            ids_hbm.at[pl.ds(0, n_batch), pl.ds(pbase, pos_per_w)], idx_v
        )

        def fire_pos(g, pg):
            pltpu.async_copy(
                pos_hbm.at[pl.ds(pbase + g * _CHUNK, _CHUNK)], pbuf.at[pg],
                psem[pg],
            )

        def wait_pos(pg):
            pltpu.make_async_copy(
                pos_hbm.at[pl.ds(0, _CHUNK)], pbuf.at[pg], psem[pg]
            ).wait()

        def fire_gather(g, b, slot):
            pltpu.async_copy(
                tok_hbm.at[idx_v.at[b, pl.ds(g * _CHUNK, _CHUNK)]],
                tbuf.at[slot], gsem[slot],
            )

        def wait_gather(slot):
            pltpu.make_async_copy(
                tok_hbm.at[pl.ds(0, _CHUNK)], tbuf.at[slot], gsem[slot]
            ).wait()

        def fire_out(g, b, slot):
            pltpu.async_copy(
                tbuf.at[slot],
                out_hbm.at[pl.ds(b * seq_len + pbase + g * _CHUNK, _CHUNK)],
                osem[slot],
            )

        def wait_out(slot):
            pltpu.make_async_copy(
                tbuf.at[slot], out_hbm.at[pl.ds(0, _CHUNK)], osem[slot]
            ).wait()

        def add_pos(slot, pg):
            def row_body(rb, c):
                for rr in range(2):
                    r = rb * 2 + rr
                    for j in range(vregs_per_row):
                        sl = pl.ds(j * _LANES, _LANES)
                        plsc.addupdate(tbuf.at[slot, r, sl], pbuf[pg, r, sl])
                return c

            lax.fori_loop(0, _CHUNK // 2, row_body, 0)

        fire_pos(0, 0)
        for b in range(_DIST):
            fire_gather(0, b, b)

        def gg_body(gg, carry):
            for g_par in (0, 1):
                g = 2 * gg + g_par
                pg = g_par
                for b in range(n_batch):
                    slot = 4 * g_par + b
                    nxt = (slot + _DIST) % _RING
                    wait_gather(slot)
                    if b == 0:
                        wait_pos(pg)
                        if g_par == 1:
                            @pl.when(gg < n_chunks // 2 - 1)
                            def _():
                                fire_pos(g + 1, 1 - pg)
                        else:
                            fire_pos(g + 1, 1 - pg)
                    if g_par == 0 or b == 0:
                        @pl.when(gg > 0)
                        def _():
                            wait_out(nxt)
                    else:
                        wait_out(nxt)
                    if b == 0:
                        fire_gather(g, 3, nxt)
                    elif g_par == 0:
                        fire_gather(g + 1, b - 1, nxt)
                    else:
                        @pl.when(gg < n_chunks // 2 - 1)
                        def _():
                            fire_gather(g + 1, b - 1, nxt)
                    add_pos(slot, pg)
                    fire_out(g, b, slot)
            return carry

        lax.fori_loop(0, n_chunks // 2, gg_body, 0)
        for slot in range(_DIST, _RING):
            wait_out(slot)

    return k


def kernel(x_ids, token_table, pos_table):
    b, t = x_ids.shape
    _, d = token_table.shape
    flat_ids = x_ids.astype(jnp.int32)
    info = plsc.get_sparse_core_info()
    n_workers = info.num_cores * info.num_subcores
    k = _embed_kernel(b, t, d, n_workers, info.num_cores)
    out = k(flat_ids, token_table, pos_table)
    return out.reshape(b, t, d)


# speedup vs baseline: 1.6147x; 1.6147x over previous
import functools

import jax
import jax.numpy as jnp
from jax import lax
from jax.experimental import pallas as pl
from jax.experimental.pallas import tpu as pltpu
from jax.experimental.pallas import tpu_sc as plsc

_LANES = 16
_CHUNK = 8
_RING = 8
_DIST = 3


def _embed_kernel(n_batch, seq_len, d_model, n_workers, n_cores):
    pos_per_w = seq_len // n_workers
    n_chunks = pos_per_w // _CHUNK
    vregs_per_row = d_model // _LANES
    assert n_chunks % 2 == 0 and n_batch == 4

    mesh = plsc.VectorSubcoreMesh(core_axis_name="c", subcore_axis_name="s")

    @functools.partial(
        pl.kernel,
        mesh=mesh,
        out_type=jax.ShapeDtypeStruct((n_batch * seq_len, d_model), jnp.float32),
        scratch_types=[
            pltpu.VMEM((n_batch, pos_per_w), jnp.int32),
            pltpu.VMEM((_RING, _CHUNK, d_model), jnp.float32),
            pltpu.VMEM((2, _CHUNK, d_model), jnp.float32),
        ]
        + [pltpu.SemaphoreType.DMA] * (2 * _RING + 2),
    )
    def k(ids_hbm, tok_hbm, pos_hbm, out_hbm, idx_v, tbuf, pbuf, *sems):
        gsem = sems[:_RING]
        osem = sems[_RING:2 * _RING]
        psem = sems[2 * _RING:]
        wid = lax.axis_index("s") * n_cores + lax.axis_index("c")
        pbase = wid * pos_per_w

        pltpu.sync_copy(
            ids_hbm.at[pl.ds(0, n_batch), pl.ds(pbase, pos_per_w)], idx_v
        )

        def fire_pos(g, pg):
            pltpu.async_copy(
                pos_hbm.at[pl.ds(pbase + g * _CHUNK, _CHUNK)], pbuf.at[pg],
                psem[pg],
            )

        def wait_pos(pg):
            pltpu.make_async_copy(
                pos_hbm.at[pl.ds(0, _CHUNK)], pbuf.at[pg], psem[pg]
            ).wait()

        def fire_gather(g, b, slot):
            pltpu.async_copy(
                tok_hbm.at[idx_v.at[b, pl.ds(g * _CHUNK, _CHUNK)]],
                tbuf.at[slot], gsem[slot],
            )

        def wait_gather(slot):
            pltpu.make_async_copy(
                tok_hbm.at[pl.ds(0, _CHUNK)], tbuf.at[slot], gsem[slot]
            ).wait()

        def fire_out(g, b, slot):
            pltpu.async_copy(
                tbuf.at[slot],
                out_hbm.at[pl.ds(b * seq_len + pbase + g * _CHUNK, _CHUNK)],
                osem[slot],
            )

        def wait_out(slot):
            pltpu.make_async_copy(
                tbuf.at[slot], out_hbm.at[pl.ds(0, _CHUNK)], osem[slot]
            ).wait()

        def add_pos(slot, pg):
            def row_body(r, c):
                for j in range(vregs_per_row):
                    sl = pl.ds(j * _LANES, _LANES)
                    plsc.addupdate(tbuf.at[slot, r, sl], pbuf[pg, r, sl])
                return c

            lax.fori_loop(0, _CHUNK, row_body, 0)

        fire_pos(0, 0)
        for b in range(_DIST):
            fire_gather(0, b, b)

        def gg_body(gg, carry):
            for g_par in (0, 1):
                g = 2 * gg + g_par
                pg = g_par
                for b in range(n_batch):
                    slot = 4 * g_par + b
                    nxt = (slot + _DIST) % _RING
                    wait_gather(slot)
                    if b == 0:
                        wait_pos(pg)
                        if g_par == 1:
                            @pl.when(gg < n_chunks // 2 - 1)
                            def _():
                                fire_pos(g + 1, 1 - pg)
                        else:
                            fire_pos(g + 1, 1 - pg)
                    if g_par == 0 or b == 0:
                        @pl.when(gg > 0)
                        def _():
                            wait_out(nxt)
                    else:
                        wait_out(nxt)
                    if b == 0:
                        fire_gather(g, 3, nxt)
                    elif g_par == 0:
                        fire_gather(g + 1, b - 1, nxt)
                    else:
                        @pl.when(gg < n_chunks // 2 - 1)
                        def _():
                            fire_gather(g + 1, b - 1, nxt)
                    add_pos(slot, pg)
                    fire_out(g, b, slot)
            return carry

        lax.fori_loop(0, n_chunks // 2, gg_body, 0)
        for slot in range(_DIST, _RING):
            wait_out(slot)

    return k


def kernel(x_ids, token_table, pos_table):
    b, t = x_ids.shape
    _, d = token_table.shape
    flat_ids = x_ids.astype(jnp.int32)
    info = plsc.get_sparse_core_info()
    n_workers = info.num_cores * info.num_subcores
    k = _embed_kernel(b, t, d, n_workers, info.num_cores)
    out = k(flat_ids, token_table, pos_table)
    return out.reshape(b, t, d)
